# Initial kernel scaffold; baseline (speedup 1.0000x reference)
import jax
import jax.numpy as jnp
from jax.experimental import pallas as pl


def kernel(token_ids, table):
    raise NotImplementedError("write your pallas kernel here")


# speedup vs baseline: 2.9714x; 2.9714x over previous
import functools

import jax
import jax.numpy as jnp
from jax import lax
from jax.experimental import pallas as pl
from jax.experimental.pallas import tpu as pltpu
from jax.experimental.pallas import tpu_sc as plsc

VOCAB = 100000
DIM = 128
BATCH = 4096
SEQ = 50
NTOK = BATCH * SEQ

NC = 2
NS = 16
NW = NC * NS
TOK_PER_W = NTOK // NW
CHUNK = 128
NCHUNK = TOK_PER_W // CHUNK

_mesh = plsc.VectorSubcoreMesh(core_axis_name="c", subcore_axis_name="s")


@functools.partial(
    pl.kernel,
    mesh=_mesh,
    out_type=jax.ShapeDtypeStruct((NTOK, DIM), jnp.float32),
    scratch_types=[
        pltpu.VMEM((NCHUNK, CHUNK), jnp.int32),
        pltpu.VMEM((CHUNK, DIM), jnp.float32),
        pltpu.SemaphoreType.DMA,
    ],
)
def _embed(ids_hbm, table_hbm, out_hbm, idx_v, buf_v, gsem):
    wid = lax.axis_index("s") * NC + lax.axis_index("c")
    base = wid * TOK_PER_W
    pltpu.sync_copy(ids_hbm.at[wid], idx_v)

    def body(g, carry):
        pltpu.async_copy(table_hbm.at[idx_v.at[g]], buf_v, gsem).wait()
        pltpu.sync_copy(buf_v, out_hbm.at[pl.ds(base + g * CHUNK, CHUNK)])
        return carry

    lax.fori_loop(0, NCHUNK, body, 0)


def kernel(token_ids, table):
    ids = token_ids.reshape(NW, NCHUNK, CHUNK).astype(jnp.int32)
    out = _embed(ids, table)
    return out.reshape(BATCH, SEQ, DIM)


# speedup vs baseline: 3.3551x; 1.1291x over previous
import functools

import jax
import jax.numpy as jnp
from jax import lax
from jax.experimental import pallas as pl
from jax.experimental.pallas import tpu as pltpu
from jax.experimental.pallas import tpu_sc as plsc

VOCAB = 100000
DIM = 128
BATCH = 4096
SEQ = 50
NTOK = BATCH * SEQ

NC = 2
NS = 16
NW = NC * NS
TOK_PER_W = NTOK // NW
CHUNK = 128
NCHUNK = TOK_PER_W // CHUNK
M = 3
NBUF = 2 * M

_mesh = plsc.VectorSubcoreMesh(core_axis_name="c", subcore_axis_name="s")


@functools.partial(
    pl.kernel,
    mesh=_mesh,
    out_type=jax.ShapeDtypeStruct((NTOK, DIM), jnp.float32),
    scratch_types=[
        pltpu.VMEM((NCHUNK, CHUNK), jnp.int32),
        pltpu.VMEM((NBUF, CHUNK, DIM), jnp.float32),
        pltpu.SemaphoreType.DMA,
        pltpu.SemaphoreType.DMA,
    ],
)
def _embed(ids_hbm, table_hbm, out_hbm, idx_v, bufs, gsem, ssem):
    wid = lax.axis_index("s") * NC + lax.axis_index("c")
    base = wid * TOK_PER_W
    pltpu.sync_copy(ids_hbm.at[wid], idx_v)

    def wait_gather(b):
        pltpu.make_async_copy(table_hbm.at[pl.ds(0, CHUNK)], bufs.at[b], gsem).wait()

    def wait_scatter():
        pltpu.make_async_copy(bufs.at[0], out_hbm.at[pl.ds(base, CHUNK)], ssem).wait()

    def gather(g, b):
        pltpu.async_copy(table_hbm.at[idx_v.at[g]], bufs.at[b], gsem)

    def scatter(g, b):
        pltpu.async_copy(bufs.at[b], out_hbm.at[pl.ds(base + g * CHUNK, CHUNK)], ssem)

    for b in range(M):
        gather(b, b)
    for g in range(M):
        wait_gather(g)
        scatter(g, g)
        gather(g + M, (g + M) % NBUF)
    def body(g, carry):
        b = lax.rem(g, NBUF)
        wait_gather(b)
        scatter(g, b)
        wait_scatter()
        gather(g + M, lax.rem(g + M, NBUF))
        return carry

    lax.fori_loop(M, NCHUNK - M, body, 0)
    for g in range(NCHUNK - M, NCHUNK):
        wait_gather(g % NBUF)
        scatter(g, g % NBUF)
    for _ in range(NBUF):
        wait_scatter()


def kernel(token_ids, table):
    ids = token_ids.reshape(NW, NCHUNK, CHUNK).astype(jnp.int32)
    out = _embed(ids, table)
    return out.reshape(BATCH, SEQ, DIM)
